# Initial kernel scaffold; baseline (speedup 1.0000x reference)
import jax
import jax.numpy as jnp
from jax.experimental import pallas as pl


def kernel(hidden_states, gate_weight, expert_gate_w, expert_up_w, expert_down_w, shared_gate_w, shared_up_w, shared_down_w):
    raise NotImplementedError("write your pallas kernel here")


# speedup vs baseline: 4.9338x; 4.9338x over previous
import functools

import jax
import jax.numpy as jnp
from jax.experimental import pallas as pl
from jax.experimental.pallas import tpu as pltpu


def _moe_body(n_shared_tiles, h_ref, gw_ref, wg_ref, wu_ref, wd_ref,
              sg_ref, su_ref, sd_ref, out_ref, scores_ref):
    e = pl.program_id(0)
    j = pl.program_id(1)

    @pl.when((e == 0) & (j == 0))
    def _init():
        h = h_ref[...]
        logits = jax.lax.dot_general(
            h, gw_ref[...], (((1,), (1,)), ((), ())),
            preferred_element_type=jnp.float32)
        mx = jnp.max(logits, axis=1, keepdims=True)
        ex = jnp.exp(logits - mx)
        scores_ref[...] = ex / jnp.sum(ex, axis=1, keepdims=True)
        out_ref[...] = jnp.zeros_like(out_ref)

    s = scores_ref[...]
    col = jax.lax.broadcasted_iota(jnp.int32, s.shape, 1)
    onehot = (col == e).astype(s.dtype)
    se = jnp.sum(s * onehot, axis=1, keepdims=True)
    beats = (s > se) | ((s == se) & (col < e))
    cnt = jnp.sum(beats.astype(jnp.int32), axis=1, keepdims=True)
    m = jnp.where(cnt < 2, se, 0.0)

    h = h_ref[...]
    g = jnp.dot(h, wg_ref[0], preferred_element_type=jnp.float32)
    u = jnp.dot(h, wu_ref[0], preferred_element_type=jnp.float32)
    t = g * jax.nn.sigmoid(g) * u
    part = jnp.dot(t, wd_ref[0], preferred_element_type=jnp.float32)
    out_ref[...] += m * part

    @pl.when((e == 0) & (j < n_shared_tiles))
    def _shared():
        sg = jnp.dot(h, sg_ref[...], preferred_element_type=jnp.float32)
        su = jnp.dot(h, su_ref[...], preferred_element_type=jnp.float32)
        st = sg * jax.nn.sigmoid(sg) * su
        out_ref[...] += jnp.dot(st, sd_ref[...], preferred_element_type=jnp.float32)


def kernel(hidden_states, gate_weight, expert_gate_w, expert_up_w, expert_down_w,
           shared_gate_w, shared_up_w, shared_down_w):
    B, S, D = hidden_states.shape
    N = B * S
    E = gate_weight.shape[0]
    FF = expert_gate_w.shape[2]
    SFF = shared_gate_w.shape[1]
    TF = 512
    assert FF % TF == 0 and SFF % TF == 0
    nj = FF // TF
    nsh = SFF // TF

    h = hidden_states.reshape(N, D)

    grid = (E, nj)
    out = pl.pallas_call(
        functools.partial(_moe_body, nsh),
        grid=grid,
        in_specs=[
            pl.BlockSpec((N, D), lambda e, j: (0, 0)),
            pl.BlockSpec((E, D), lambda e, j: (0, 0)),
            pl.BlockSpec((1, D, TF), lambda e, j: (e, 0, j)),
            pl.BlockSpec((1, D, TF), lambda e, j: (e, 0, j)),
            pl.BlockSpec((1, TF, D), lambda e, j: (e, j, 0)),
            pl.BlockSpec((D, TF), lambda e, j: (0, jnp.minimum(j, nsh - 1))),
            pl.BlockSpec((D, TF), lambda e, j: (0, jnp.minimum(j, nsh - 1))),
            pl.BlockSpec((TF, D), lambda e, j: (jnp.minimum(j, nsh - 1), 0)),
        ],
        out_specs=pl.BlockSpec((N, D), lambda e, j: (0, 0)),
        out_shape=jax.ShapeDtypeStruct((N, D), jnp.float32),
        scratch_shapes=[pltpu.VMEM((N, E), jnp.float32)],
        compiler_params=pltpu.CompilerParams(
            dimension_semantics=("arbitrary", "arbitrary"),
        ),
    )(h, gate_weight, expert_gate_w, expert_up_w, expert_down_w,
      shared_gate_w, shared_up_w, shared_down_w)

    return out.reshape(B, S, D)


# speedup vs baseline: 5.0157x; 1.0166x over previous
import functools

import jax
import jax.numpy as jnp
from jax.experimental import pallas as pl
from jax.experimental.pallas import tpu as pltpu


def _moe_body(n_shared_tiles, h_ref, gw_ref, wg_ref, wu_ref, wd_ref,
              sg_ref, su_ref, sd_ref, out_ref, scores_ref):
    e = pl.program_id(0)
    j = pl.program_id(1)

    @pl.when((e == 0) & (j == 0))
    def _init():
        h = h_ref[...]
        logits = jax.lax.dot_general(
            h, gw_ref[...], (((1,), (1,)), ((), ())),
            preferred_element_type=jnp.float32)
        mx = jnp.max(logits, axis=1, keepdims=True)
        ex = jnp.exp(logits - mx)
        scores_ref[...] = ex / jnp.sum(ex, axis=1, keepdims=True)
        out_ref[...] = jnp.zeros_like(out_ref)

    s = scores_ref[...]
    col = jax.lax.broadcasted_iota(jnp.int32, s.shape, 1)
    onehot = (col == e).astype(s.dtype)
    se = jnp.sum(s * onehot, axis=1, keepdims=True)
    beats = (s > se) | ((s == se) & (col < e))
    cnt = jnp.sum(beats.astype(jnp.int32), axis=1, keepdims=True)
    m = jnp.where(cnt < 2, se, 0.0)

    h = h_ref[...].astype(jnp.bfloat16)
    g = jnp.dot(h, wg_ref[0].astype(jnp.bfloat16), preferred_element_type=jnp.float32)
    u = jnp.dot(h, wu_ref[0].astype(jnp.bfloat16), preferred_element_type=jnp.float32)
    t = (g * jax.nn.sigmoid(g) * u).astype(jnp.bfloat16)
    part = jnp.dot(t, wd_ref[0].astype(jnp.bfloat16), preferred_element_type=jnp.float32)
    out_ref[...] += m * part

    @pl.when((e == 0) & (j < n_shared_tiles))
    def _shared():
        sg = jnp.dot(h, sg_ref[...].astype(jnp.bfloat16), preferred_element_type=jnp.float32)
        su = jnp.dot(h, su_ref[...].astype(jnp.bfloat16), preferred_element_type=jnp.float32)
        st = (sg * jax.nn.sigmoid(sg) * su).astype(jnp.bfloat16)
        out_ref[...] += jnp.dot(st, sd_ref[...].astype(jnp.bfloat16), preferred_element_type=jnp.float32)


def kernel(hidden_states, gate_weight, expert_gate_w, expert_up_w, expert_down_w,
           shared_gate_w, shared_up_w, shared_down_w):
    B, S, D = hidden_states.shape
    N = B * S
    E = gate_weight.shape[0]
    FF = expert_gate_w.shape[2]
    SFF = shared_gate_w.shape[1]
    TF = 512
    assert FF % TF == 0 and SFF % TF == 0
    nj = FF // TF
    nsh = SFF // TF

    h = hidden_states.reshape(N, D)

    grid = (E, nj)
    out = pl.pallas_call(
        functools.partial(_moe_body, nsh),
        grid=grid,
        in_specs=[
            pl.BlockSpec((N, D), lambda e, j: (0, 0)),
            pl.BlockSpec((E, D), lambda e, j: (0, 0)),
            pl.BlockSpec((1, D, TF), lambda e, j: (e, 0, j)),
            pl.BlockSpec((1, D, TF), lambda e, j: (e, 0, j)),
            pl.BlockSpec((1, TF, D), lambda e, j: (e, j, 0)),
            pl.BlockSpec((D, TF), lambda e, j: (0, jnp.minimum(j, nsh - 1))),
            pl.BlockSpec((D, TF), lambda e, j: (0, jnp.minimum(j, nsh - 1))),
            pl.BlockSpec((TF, D), lambda e, j: (jnp.minimum(j, nsh - 1), 0)),
        ],
        out_specs=pl.BlockSpec((N, D), lambda e, j: (0, 0)),
        out_shape=jax.ShapeDtypeStruct((N, D), jnp.float32),
        scratch_shapes=[pltpu.VMEM((N, E), jnp.float32)],
        compiler_params=pltpu.CompilerParams(
            dimension_semantics=("arbitrary", "arbitrary"),
        ),
    )(h, gate_weight, expert_gate_w, expert_up_w, expert_down_w,
      shared_gate_w, shared_up_w, shared_down_w)

    return out.reshape(B, S, D)
